# baseline (speedup 1.0000x reference)
import functools

import jax
import jax.numpy as jnp
from jax import lax
from jax.experimental import pallas as pl
from jax.experimental.pallas import tpu as pltpu
from jax.experimental.pallas import tpu_sc as plsc

B = 16384
D = 16
CAP = 100000
INV_GRID = 100000.0
RPB = 8
NBLK = CAP // RPB

_info = plsc.get_sparse_core_info()
_NC, _NS, _L = _info.num_cores, _info.num_subcores, _info.num_lanes
NW = _NC * _NS
BPW = B // NW
CHUNK = 128
NCH = BPW // CHUNK
NGRP = CHUNK // 16


@functools.partial(
    pl.kernel,
    out_type=jax.ShapeDtypeStruct((B, 2 * D), jnp.float32),
    mesh=plsc.VectorSubcoreMesh(core_axis_name="c", subcore_axis_name="s"),
    compiler_params=pltpu.CompilerParams(needs_layout_passes=False),
    scratch_types=[
        pltpu.VMEM((2, BPW), jnp.float32),
        pltpu.VMEM((2, NCH, CHUNK), jnp.int32),
        pltpu.VMEM((2, NCH, CHUNK), jnp.int32),
        pltpu.VMEM((2, 2, CHUNK, 2 * D * 4), jnp.float32),
        pltpu.VMEM((2, CHUNK, 2 * D), jnp.float32),
        pltpu.SemaphoreType.DMA,
        pltpu.SemaphoreType.DMA,
    ],
)
def _grid_gather(obs_t, t0c, t1c, out, obs_v, idx_v, off_v, blk_v, o_v,
                 gsem, osem):
    wid = lax.axis_index("s") * _NC + lax.axis_index("c")
    base = wid * BPW
    pltpu.sync_copy(obs_t.at[0, pl.ds(base, BPW)], obs_v.at[0])
    pltpu.sync_copy(obs_t.at[1, pl.ds(base, BPW)], obs_v.at[1])
    for f in range(2):
        for c in range(NCH):
            for j in range(CHUNK // _L):
                x = obs_v[f, pl.ds(c * CHUNK + j * _L, _L)]
                r = (x * INV_GRID).astype(jnp.int32)
                r = jnp.minimum(jnp.maximum(r, 0), CAP - 1)
                idx_v[f, c, pl.ds(j * _L, _L)] = r >> 3
                off_v[f, c, pl.ds(j * _L, _L)] = (r & 7) << 4

    def fire(c, slot):
        return [pltpu.async_copy(tbl.at[idx_v.at[f, c]], blk_v.at[slot, f],
                                 gsem)
                for f, tbl in ((0, t0c), (1, t1c))]

    pending = fire(0, 0)
    for c in range(NCH):
        slot = c % 2
        for cp in pending:
            cp.wait()
        if c + 1 < NCH:
            pending = fire(c + 1, 1 - slot)
        rows = lax.iota(jnp.int32, _L)
        for f in range(2):
            for g in range(NGRP):
                grows = rows + g * 16
                cols = off_v[f, c, pl.ds(g * 16, 16)]
                for e in range(D):
                    vals = plsc.load_gather(blk_v.at[slot, f],
                                            [grows, cols + e])
                    plsc.store_scatter(o_v.at[slot], [grows,
                                       jnp.full((16,), f * D + e, jnp.int32)],
                                       vals)
        pltpu.async_copy(o_v.at[slot],
                         out.at[pl.ds(base + c * CHUNK, CHUNK)], osem).wait()


def kernel(obs, table0, table1):
    obs_t = obs.T
    t0c = table0.reshape(NBLK, RPB * D)
    t1c = table1.reshape(NBLK, RPB * D)
    return _grid_gather(obs_t, t0c, t1c)
